# baseline (speedup 1.0000x reference)
import jax
import jax.numpy as jnp
from jax import lax
from jax.experimental import pallas as pl
from jax.experimental.pallas import tpu as pltpu
from jax.experimental.pallas import tpu_sc as plsc

_NC = 2
_NS = 16
_NW = _NC * _NS
_ACC_MAX = 448


def _plan_passes(n_chunks, p):
    if n_chunks * p <= _ACC_MAX:
        passes = [(0, n_chunks)]
    else:
        chz = max((_ACC_MAX // 2) // p // 2 * 2, 2)
        s1 = max((n_chunks // 2) // chz * chz, 2)
        passes = [(0, s1), (s1, n_chunks - s1)]
        for _, nch in passes:
            assert nch % 2 == 0
    for _, nch in passes:
        assert nch >= 2
    acc_rows = max(nch for _, nch in passes) * p
    return passes, acc_rows


def _zrows(rows, pk):
    for d in (4, 2, 1):
        if rows % d == 0 and rows // d <= pk:
            return rows // d
    raise AssertionError("no zero-init granule")


def _make_gather_sum(ntp, c, k, p):
    pk = p * k
    n_per_w = ntp // _NW
    n_chunks = n_per_w // p
    assert ntp % (_NW * p) == 0 and c % 16 == 0
    passes, acc_rows = _plan_passes(n_chunks, p)

    mesh = plsc.VectorSubcoreMesh(
        core_axis_name="c", subcore_axis_name="s",
        num_cores=_NC, num_subcores=_NS)

    def body(table, idx, dst, out, idx_v, dst_v, buf_a, buf_b, acc,
             gs_a, gs_b, as_a, as_b):
        s = lax.axis_index("s")
        wid = s * _NC + lax.axis_index("c")
        row0 = wid * n_per_w
        sbase = s * acc_rows
        pltpu.sync_copy(idx.at[wid], idx_v)
        pltpu.sync_copy(dst.at[s], dst_v)

        def fire_gather(ch, buf, sem):
            pltpu.async_copy(table.at[idx_v.at[ch]], buf, sem)

        def wait_gather(buf, sem):
            pltpu.make_async_copy(table.at[pl.ds(0, pk)], buf, sem).wait()

        for ch0, nch in passes:
            zr = _zrows(nch * p, pk)
            z = jnp.zeros((16,), jnp.float32)
            for r in range(zr):
                for cc in range(0, c, 16):
                    buf_a[r, pl.ds(cc, 16)] = z
            for t in range((nch * p) // zr):
                pltpu.sync_copy(buf_a.at[pl.ds(0, zr)],
                                acc.at[pl.ds(sbase + t * zr, zr)])

            fire_gather(ch0, buf_a, gs_a)
            fire_gather(ch0 + 1, buf_b, gs_b)

            def step(i, carry):
                c0 = ch0 + 2 * i
                wait_gather(buf_a, gs_a)
                pltpu.async_copy(buf_a, acc.at[dst_v.at[c0]], as_a,
                                 add=True).wait()

                @pl.when(c0 + 2 < ch0 + nch)
                def _():
                    fire_gather(c0 + 2, buf_a, gs_a)

                c1 = c0 + 1
                wait_gather(buf_b, gs_b)
                pltpu.async_copy(buf_b, acc.at[dst_v.at[c1]], as_b,
                                 add=True).wait()

                @pl.when(c1 + 2 < ch0 + nch)
                def _():
                    fire_gather(c1 + 2, buf_b, gs_b)

                return carry

            lax.fori_loop(0, nch // 2, step, 0)
            if nch % 2:
                cl = ch0 + nch - 1
                wait_gather(buf_a, gs_a)
                pltpu.async_copy(buf_a, acc.at[dst_v.at[cl]], as_a,
                                 add=True).wait()
            pltpu.sync_copy(acc.at[pl.ds(sbase, nch * p)],
                            out.at[pl.ds(row0 + ch0 * p, nch * p)])

    return pl.kernel(
        body,
        out_type=jax.ShapeDtypeStruct((ntp, c), jnp.float32),
        mesh=mesh,
        scratch_types=[
            pltpu.VMEM((n_chunks, pk), jnp.int32),
            pltpu.VMEM((n_chunks, pk), jnp.int32),
            pltpu.VMEM((pk, c), jnp.float32),
            pltpu.VMEM((pk, c), jnp.float32),
            pltpu.VMEM_SHARED((_NS * acc_rows, c), jnp.float32),
            pltpu.SemaphoreType.DMA,
            pltpu.SemaphoreType.DMA,
            pltpu.SemaphoreType.DMA,
            pltpu.SemaphoreType.DMA,
        ],
    )


def _make_dst3d(ntp, k, p):
    n_chunks = ntp // _NW // p
    passes, acc_rows = _plan_passes(n_chunks, p)
    ch = jnp.arange(n_chunks, dtype=jnp.int32)
    lch = ch
    for ch0, _ in passes[1:]:
        lch = jnp.where(ch >= ch0, ch - ch0, lch)
    return (jnp.arange(_NS, dtype=jnp.int32)[:, None, None] * acc_rows
            + lch[None, :, None] * p
            + jnp.repeat(jnp.arange(p, dtype=jnp.int32), k)[None, None, :])


def _relu_transpose(points_p, np_, c, nb):
    b = points_p.shape[0]

    def body(x_ref, o_ref):
        o_ref[0] = jnp.maximum(x_ref[0], 0.0).T

    return pl.pallas_call(
        body,
        grid=(b, np_ // nb),
        in_specs=[pl.BlockSpec((1, c, nb), lambda i, j: (i, 0, j))],
        out_specs=pl.BlockSpec((1, nb, c), lambda i, j: (i, j, 0)),
        out_shape=jax.ShapeDtypeStruct((b, np_, c), jnp.float32),
    )(points_p)


def _mix(points_p, g1, w1, w2, np_, c, nb, k):
    b = points_p.shape[0]
    inv = 1.0 / (k + 1.0)

    def body(x_ref, g_ref, w1_ref, w2_ref, o_ref):
        x = x_ref[0]
        y1 = lax.dot_general(jnp.maximum(x, 0.0), w1_ref[...],
                             (((0,), (1,)), ((), ())),
                             preferred_element_type=jnp.float32)
        y2 = lax.dot_general(g_ref[0], w2_ref[...],
                             (((1,), (1,)), ((), ())),
                             preferred_element_type=jnp.float32)
        o_ref[0] = jnp.maximum((y1 + y2) * inv + x.T, 0.0)

    return pl.pallas_call(
        body,
        grid=(b, np_ // nb),
        in_specs=[
            pl.BlockSpec((1, c, nb), lambda i, j: (i, 0, j)),
            pl.BlockSpec((1, nb, c), lambda i, j: (i, j, 0)),
            pl.BlockSpec((c, c), lambda i, j: (0, 0)),
            pl.BlockSpec((c, c), lambda i, j: (0, 0)),
        ],
        out_specs=pl.BlockSpec((1, nb, c), lambda i, j: (i, j, 0)),
        out_shape=jax.ShapeDtypeStruct((b, np_, c), jnp.float32),
    )(points_p, g1, w1, w2)


def _project(g2, wup_p, xyz12, np_, c, nb, k, oc):
    b = g2.shape[0]
    inv = 1.0 / k

    def body(g_ref, w_ref, x_ref, o_ref):
        y = lax.dot_general(g_ref[0], w_ref[...],
                            (((1,), (1,)), ((), ())),
                            preferred_element_type=jnp.float32)
        o_ref[0] = y * inv + x_ref[0]

    return pl.pallas_call(
        body,
        grid=(b, np_ // nb),
        in_specs=[
            pl.BlockSpec((1, nb, c), lambda i, j: (i, j, 0)),
            pl.BlockSpec((oc, c), lambda i, j: (0, 0)),
            pl.BlockSpec((1, nb, oc), lambda i, j: (i, j, 0)),
        ],
        out_specs=pl.BlockSpec((1, nb, oc), lambda i, j: (i, j, 0)),
        out_shape=jax.ShapeDtypeStruct((b, np_, oc), jnp.float32),
    )(g2, wup_p, xyz12)


def kernel(xyz, points, indices, W1, W2, Wup):
    b, c, n = points.shape
    k = indices.shape[2]
    oc = Wup.shape[0]
    up = oc // 3

    nb = 896
    np_ = ((n + nb - 1) // nb) * nb
    pad = np_ - n
    p = 128 // k
    assert np_ % (_NW * p) == 0

    points_p = jnp.pad(points, ((0, 0), (0, 0), (0, pad)))
    idx_p = jnp.pad(indices, ((0, 0), (0, pad), (0, 0)))
    idx3d = idx_p.reshape(b, _NW, np_ // (_NW * p), p * k)
    dst3d = _make_dst3d(np_, k, p)

    gather_sum = _make_gather_sum(np_, c, k, p)

    wup_p = Wup.reshape(3, up, c).transpose(1, 0, 2).reshape(oc, c)
    xyz_p = jnp.pad(xyz, ((0, 0), (0, pad), (0, 0)))
    xyz12 = jnp.tile(xyz_p, (1, 1, up))

    outs = []
    for i in range(b):
        pts_i = points_p[i:i + 1]
        feats_t = _relu_transpose(pts_i, np_, c, nb)
        g1 = gather_sum(feats_t.reshape(np_, c), idx3d[i], dst3d)
        f_t = _mix(pts_i, g1.reshape(1, np_, c), W1, W2, np_, c, nb, k)
        g2 = gather_sum(f_t.reshape(np_, c), idx3d[i], dst3d)
        out12 = _project(g2.reshape(1, np_, c), wup_p, xyz12[i:i + 1],
                         np_, c, nb, k, oc)
        outs.append(out12[:, :n, :])
    return jnp.concatenate(outs, axis=0).reshape(b, n * up, 3)
